# baseline (speedup 1.0000x reference)
import functools

import jax
import jax.numpy as jnp
from jax.experimental import pallas as pl
from jax.experimental.pallas import tpu as pltpu

ALPHA = 0.3
BETA = 0.7
GAMMA = 1.33
SMOOTH = 1e-06
LOVASZ_WEIGHT = 0.2

_R = 2048
_C = 128
_N = _R * _C
_LOGN = 18
_CHR = 64
_CH = _CHR * _C
_LOGCH = 13
_NCH = _N // _CH


def _stage(x, row_pos, lane_pos, j, dbit):
    if j >= _C:
        axis, amt, pos = 0, j // _C, row_pos
    else:
        axis, amt, pos = 1, j, lane_pos
    fwd = jnp.roll(x, -amt, axis)
    bwd = jnp.roll(x, amt, axis)
    lb = (pos >> (amt.bit_length() - 1)) & 1
    partner = jnp.where(lb == 0, fwd, bwd)
    keep_max = dbit == lb
    return jnp.where((x < partner) == keep_max, partner, x)


def _dir_bit(k, row_pos, lane_pos, chunk_idx):
    if k < 7:
        return (lane_pos >> k) & 1
    if k < _LOGCH:
        return (row_pos >> (k - 7)) & 1
    return (chunk_idx >> (k - _LOGCH)) & 1


def _local_init_kernel(x_ref, t_ref, key_ref, part_ref):
    c = pl.program_id(1)
    x = x_ref[0]
    t = t_ref[0].astype(jnp.float32)
    p = jax.nn.sigmoid(x)
    part_ref[0, 0, 0, :] = jnp.sum(p * t, axis=0)
    part_ref[0, 0, 1, :] = jnp.sum((1.0 - p) * t, axis=0)
    part_ref[0, 0, 2, :] = jnp.sum(p * (1.0 - t), axis=0)

    err = 1.0 - x * (2.0 * t - 1.0)
    b = jax.lax.bitcast_convert_type(err, jnp.int32)
    key = b ^ jax.lax.shift_right_logical(
        jax.lax.shift_right_arithmetic(b, 31), 1)
    key = (key & ~jnp.int32(1)) | t_ref[0].astype(jnp.int32)

    row_pos = jax.lax.broadcasted_iota(jnp.int32, (_CHR, _C), 0)
    lane_pos = jax.lax.broadcasted_iota(jnp.int32, (_CHR, _C), 1)
    for k in range(1, _LOGCH + 1):
        db = _dir_bit(k, row_pos, lane_pos, c)
        for lj in range(k - 1, -1, -1):
            key = _stage(key, row_pos, lane_pos, 1 << lj, db)
    key_ref[0] = key


def _finish_kernel(key_ref, out_ref):
    row_pos = jax.lax.broadcasted_iota(jnp.int32, (_R, _C), 0)
    lane_pos = jax.lax.broadcasted_iota(jnp.int32, (_R, _C), 1)
    key = key_ref[0]
    for k in range(_LOGCH + 1, _LOGN + 1):
        db = (row_pos >> (k - 7)) & 1
        for lj in range(k - 1, -1, -1):
            key = _stage(key, row_pos, lane_pos, 1 << lj, db)

    lab = (key & 1).astype(jnp.float32)
    kb = key & ~jnp.int32(1)
    kb = kb ^ jax.lax.shift_right_logical(
        jax.lax.shift_right_arithmetic(kb, 31), 1)
    errs = jax.lax.bitcast_convert_type(kb, jnp.float32)

    cum = lab
    for s in (1, 2, 4, 8, 16, 32, 64):
        cum = cum + jnp.where(lane_pos >= s, jnp.roll(cum, s, 1), 0.0)
    row_tot = cum[:, _C - 1:_C]
    carry = jnp.broadcast_to(row_tot, (_R, _C))
    for s in (1, 2, 4, 8, 16, 32, 64, 128, 256, 512, 1024):
        carry = carry + jnp.where(row_pos >= s, jnp.roll(carry, s, 0), 0.0)
    p_tot = jnp.max(carry)
    cum = cum + (carry - jnp.broadcast_to(row_tot, (_R, _C)))

    pos1 = (row_pos * _C + lane_pos + 1).astype(jnp.float32)
    union = p_tot + pos1 - cum
    jaccard = 1.0 - (p_tot - cum) / union
    prev = jnp.where(lane_pos == 0,
                     jnp.roll(jnp.roll(jaccard, 1, 1), 1, 0),
                     jnp.roll(jaccard, 1, 1))
    jd = jnp.where(pos1 == 1.0, jaccard, jaccard - prev)
    has_neg = (p_tot < jnp.float32(_N)).astype(jnp.float32)
    grad = jaccard + has_neg * (jd - jaccard)
    out_ref[0, 0, :] = jnp.sum(jax.nn.relu(errs) * grad, axis=0)


def _sorted_keys(x, t):
    B = x.shape[0]
    chunk_spec = pl.BlockSpec((1, _CHR, _C), lambda i, c: (i, c, 0))
    keys, parts = pl.pallas_call(
        _local_init_kernel,
        grid=(B, _NCH),
        in_specs=[chunk_spec, chunk_spec],
        out_specs=[chunk_spec,
                   pl.BlockSpec((1, 1, 8, _C), lambda i, c: (i, c, 0, 0))],
        out_shape=[jax.ShapeDtypeStruct((B, _NCH * _CHR, _C), jnp.int32),
                   jax.ShapeDtypeStruct((B, _NCH, 8, _C), jnp.float32)],
        compiler_params=pltpu.CompilerParams(
            dimension_semantics=("parallel", "arbitrary")),
    )(x.reshape(B, _NCH * _CHR, _C), t.reshape(B, _NCH * _CHR, _C))

    return keys, parts


def kernel(logits, targets):
    B = logits.shape[0]
    x = logits.reshape(B, _R, _C)
    t = targets.reshape(B, _R, _C)
    keys, parts = _sorted_keys(x, t)
    lov = pl.pallas_call(
        _finish_kernel,
        grid=(B,),
        in_specs=[pl.BlockSpec((1, _R, _C), lambda i: (i, 0, 0))],
        out_specs=pl.BlockSpec((1, 1, _C), lambda i: (i, 0, 0)),
        out_shape=jax.ShapeDtypeStruct((B, 1, _C), jnp.float32),
        compiler_params=pltpu.CompilerParams(
            dimension_semantics=("parallel",)),
    )(keys)
    tp = parts[:, :, 0, :].sum(axis=(1, 2))
    fn = parts[:, :, 1, :].sum(axis=(1, 2))
    fp = parts[:, :, 2, :].sum(axis=(1, 2))
    tversky = (tp + SMOOTH) / (tp + ALPHA * fn + BETA * fp + SMOOTH)
    ft = jnp.mean((1.0 - tversky) ** GAMMA)
    return ft + LOVASZ_WEIGHT * jnp.mean(lov.sum(axis=(1, 2)))


# speedup vs baseline: 1.0612x; 1.0612x over previous
import jax
import jax.numpy as jnp
from jax.experimental import pallas as pl
from jax.experimental.pallas import tpu as pltpu

ALPHA = 0.3
BETA = 0.7
GAMMA = 1.33
SMOOTH = 1e-06
LOVASZ_WEIGHT = 0.2

_R = 2048
_C = 128
_N = _R * _C
_LOGN = 18
_CHR = 64
_CH = _CHR * _C
_LOGCH = 13
_NCH = _N // _CH


def _stage(x, row_pos, lane_pos, j, dbit):
    if j >= _C:
        axis, amt, pos = 0, j // _C, row_pos
    else:
        axis, amt, pos = 1, j, lane_pos
    fwd = jnp.roll(x, -amt, axis)
    bwd = jnp.roll(x, amt, axis)
    lb = (pos >> (amt.bit_length() - 1)) & 1
    partner = jnp.where(lb == 0, fwd, bwd)
    keep_max = dbit == lb
    return jnp.where(keep_max, jnp.maximum(x, partner), jnp.minimum(x, partner))


def _dir_bit(k, row_pos, lane_pos, chunk_idx):
    if k < 7:
        return (lane_pos >> k) & 1
    if k < _LOGCH:
        return (row_pos >> (k - 7)) & 1
    return (chunk_idx >> (k - _LOGCH)) & 1


def _local_init_kernel(x_ref, t_ref, key_ref, part_ref):
    c = pl.program_id(1)
    x = x_ref[0]
    t = t_ref[0].astype(jnp.float32)
    p = jax.nn.sigmoid(x)
    part_ref[0, 0, 0, :] = jnp.sum(p * t, axis=0)
    part_ref[0, 0, 1, :] = jnp.sum((1.0 - p) * t, axis=0)
    part_ref[0, 0, 2, :] = jnp.sum(p * (1.0 - t), axis=0)

    err = 1.0 - x * (2.0 * t - 1.0)
    b = jax.lax.bitcast_convert_type(err, jnp.int32)
    key = b ^ jax.lax.shift_right_logical(
        jax.lax.shift_right_arithmetic(b, 31), 1)
    key = (key & ~jnp.int32(1)) | t_ref[0].astype(jnp.int32)

    row_pos = jax.lax.broadcasted_iota(jnp.int32, (_CHR, _C), 0)
    lane_pos = jax.lax.broadcasted_iota(jnp.int32, (_CHR, _C), 1)
    for k in range(1, _LOGCH + 1):
        db = _dir_bit(k, row_pos, lane_pos, c)
        for lj in range(k - 1, -1, -1):
            key = _stage(key, row_pos, lane_pos, 1 << lj, db)
    key_ref[0] = key


def _finish_kernel(key_ref, out_ref):
    row_pos = jax.lax.broadcasted_iota(jnp.int32, (_R, _C), 0)
    lane_pos = jax.lax.broadcasted_iota(jnp.int32, (_R, _C), 1)
    key = key_ref[0]
    for k in range(_LOGCH + 1, _LOGN + 1):
        db = (row_pos >> (k - 7)) & 1
        for lj in range(k - 1, -1, -1):
            key = _stage(key, row_pos, lane_pos, 1 << lj, db)

    lab = (key & 1).astype(jnp.float32)
    kb = key & ~jnp.int32(1)
    kb = kb ^ jax.lax.shift_right_logical(
        jax.lax.shift_right_arithmetic(kb, 31), 1)
    errs = jax.lax.bitcast_convert_type(kb, jnp.float32)

    cum = lab
    for s in (1, 2, 4, 8, 16, 32, 64):
        cum = cum + jnp.where(lane_pos >= s, jnp.roll(cum, s, 1), 0.0)
    row_tot = cum[:, _C - 1:_C]
    carry = jnp.broadcast_to(row_tot, (_R, _C))
    for s in (1, 2, 4, 8, 16, 32, 64, 128, 256, 512, 1024):
        carry = carry + jnp.where(row_pos >= s, jnp.roll(carry, s, 0), 0.0)
    p_tot = jnp.max(carry)
    cum = cum + (carry - jnp.broadcast_to(row_tot, (_R, _C)))

    pos1 = (row_pos * _C + lane_pos + 1).astype(jnp.float32)
    union = p_tot + pos1 - cum
    jaccard = 1.0 - (p_tot - cum) / union
    prev = jnp.where(lane_pos == 0,
                     jnp.roll(jnp.roll(jaccard, 1, 1), 1, 0),
                     jnp.roll(jaccard, 1, 1))
    jd = jnp.where(pos1 == 1.0, jaccard, jaccard - prev)
    has_neg = (p_tot < jnp.float32(_N)).astype(jnp.float32)
    grad = jaccard + has_neg * (jd - jaccard)
    out_ref[0, 0, :] = jnp.sum(jax.nn.relu(errs) * grad, axis=0)


def _sorted_keys(x, t):
    B = x.shape[0]
    chunk_spec = pl.BlockSpec((1, _CHR, _C), lambda i, c: (i, c, 0))
    keys, parts = pl.pallas_call(
        _local_init_kernel,
        grid=(B, _NCH),
        in_specs=[chunk_spec, chunk_spec],
        out_specs=[chunk_spec,
                   pl.BlockSpec((1, 1, 8, _C), lambda i, c: (i, c, 0, 0))],
        out_shape=[jax.ShapeDtypeStruct((B, _NCH * _CHR, _C), jnp.int32),
                   jax.ShapeDtypeStruct((B, _NCH, 8, _C), jnp.float32)],
        compiler_params=pltpu.CompilerParams(
            dimension_semantics=("parallel", "arbitrary")),
    )(x.reshape(B, _NCH * _CHR, _C), t.reshape(B, _NCH * _CHR, _C))

    return keys, parts


def kernel(logits, targets):
    B = logits.shape[0]
    x = logits.reshape(B, _R, _C)
    t = targets.reshape(B, _R, _C)
    keys, parts = _sorted_keys(x, t)
    lov = pl.pallas_call(
        _finish_kernel,
        grid=(B,),
        in_specs=[pl.BlockSpec((1, _R, _C), lambda i: (i, 0, 0))],
        out_specs=pl.BlockSpec((1, 1, _C), lambda i: (i, 0, 0)),
        out_shape=jax.ShapeDtypeStruct((B, 1, _C), jnp.float32),
        compiler_params=pltpu.CompilerParams(
            dimension_semantics=("parallel",)),
    )(keys)
    tp = parts[:, :, 0, :].sum(axis=(1, 2))
    fn = parts[:, :, 1, :].sum(axis=(1, 2))
    fp = parts[:, :, 2, :].sum(axis=(1, 2))
    tversky = (tp + SMOOTH) / (tp + ALPHA * fn + BETA * fp + SMOOTH)
    ft = jnp.mean((1.0 - tversky) ** GAMMA)
    return ft + LOVASZ_WEIGHT * jnp.mean(lov.sum(axis=(1, 2)))
